# baseline (speedup 1.0000x reference)
import functools

import jax
import jax.numpy as jnp
from jax import lax
from jax.experimental import pallas as pl
from jax.experimental.pallas import tpu as pltpu
from jax.experimental.pallas import tpu_sc as plsc

_F = 26
_V = 100001
_E = 50
_B = 16384
_OUT = 128
_EPS = 1e-5
_BLK = 512
_VB = 4096
_VB2 = _VB // 2
_NVB = -(-_V // _VB)
_VH = _NVB * _VB2


def _pack_bf16_pair(p):
    u = lax.bitcast_convert_type(p, jnp.uint32) + jnp.uint32(0x8000)
    lo = lax.shift_right_logical(u[:, :64], jnp.uint32(16))
    hi = u[:, 64:] & jnp.uint32(0xFFFF0000)
    return lax.bitcast_convert_type(lo | hi, jnp.int32)


def _tc_project(tabT, W3):
    def body(t_ref, w_ref, p_ref):
        p = lax.dot_general(
            t_ref[0],
            w_ref[0],
            (((0,), (0,)), ((), ())),
            preferred_element_type=jnp.float32,
        )
        p_ref[0] = jnp.concatenate(
            [_pack_bf16_pair(p[:_VB2]), _pack_bf16_pair(p[_VB2:])], axis=1
        )

    return pl.pallas_call(
        body,
        grid=(_F, _NVB),
        in_specs=[
            pl.BlockSpec((1, _E, _VB), lambda f, v: (f, 0, v)),
            pl.BlockSpec((1, _E, _OUT), lambda f, v: (f, 0, 0)),
        ],
        out_specs=pl.BlockSpec((1, _VB2, _OUT), lambda f, v: (f, v, 0)),
        out_shape=jax.ShapeDtypeStruct((_F, _VH, _OUT), jnp.int32),
    )(tabT, W3)


def _sc_gather(p_hbm_arr, x3):
    info = plsc.get_sparse_core_info()
    nc, ns = info.num_cores, info.num_subcores
    nw = nc * ns
    bpw = _B // nw
    nch = bpw // 128

    @functools.partial(
        pl.kernel,
        mesh=plsc.VectorSubcoreMesh(core_axis_name="c", subcore_axis_name="s"),
        out_type=jax.ShapeDtypeStruct((_F, _B, _OUT), jnp.int32),
        scratch_types=[
            pltpu.VMEM((nch, 128), jnp.int32),
            pltpu.VMEM((bpw, _OUT), jnp.int32),
            pltpu.SemaphoreType.DMA,
        ],
    )
    def gather_kernel(p_hbm, x_hbm, out_hbm, idx_v, rows_v, sem):
        wid = lax.axis_index("s") * nc + lax.axis_index("c")
        base = wid * bpw

        def field_body(f, carry):
            pltpu.sync_copy(x_hbm.at[f, wid], idx_v)

            def gdesc(c):
                return pltpu.make_async_copy(
                    p_hbm.at[f].at[idx_v.at[c]],
                    rows_v.at[pl.ds(c * 128, 128)],
                    sem,
                )

            for c in range(nch):
                gdesc(c).start()
            for c in range(nch):
                gdesc(c).wait()
            pltpu.sync_copy(rows_v, out_hbm.at[f, pl.ds(base, bpw), :])
            return carry

        lax.fori_loop(0, _F, field_body, 0)

    return gather_kernel(p_hbm_arr, x3)


def _tc_sum_bn(g, par_t, gamma, beta):
    nb = _B // _BLK

    def body(g_ref, par_ref, gm_ref, bt_ref, out_ref, y_buf, acc):
        p = pl.program_id(0)
        i = pl.program_id(1)

        @pl.when(p == 0)
        def _phase0():
            w = g_ref[...]
            pt = par_ref[...]
            ylo = jnp.zeros((_BLK, 64), jnp.float32)
            yhi = jnp.zeros((_BLK, 64), jnp.float32)
            for f in range(_F):
                m = pt[:, f : f + 1] > 0
                c = jnp.where(m, w[f, :, 64:], w[f, :, :64])
                ylo = ylo + lax.bitcast_convert_type(
                    lax.shift_left(c, jnp.int32(16)), jnp.float32
                )
                yhi = yhi + lax.bitcast_convert_type(
                    c & jnp.int32(-65536), jnp.float32
                )
            y = jnp.concatenate([ylo, yhi], axis=1)
            y_buf[pl.ds(i * _BLK, _BLK), :] = y

            @pl.when(i == 0)
            def _init():
                acc[...] = jnp.zeros_like(acc)

            acc[0:1, :] = acc[0:1, :] + jnp.sum(y, axis=0, keepdims=True)
            acc[1:2, :] = acc[1:2, :] + jnp.sum(y * y, axis=0, keepdims=True)

        @pl.when(p == 1)
        def _phase1():
            mean = acc[0:1, :] * (1.0 / _B)
            var = acc[1:2, :] * (1.0 / _B) - mean * mean
            scale = gm_ref[...] * lax.rsqrt(var + _EPS)
            shift = bt_ref[...] - mean * scale
            out_ref[...] = y_buf[pl.ds(i * _BLK, _BLK), :] * scale + shift

    return pl.pallas_call(
        body,
        grid=(2, nb),
        in_specs=[
            pl.BlockSpec((_F, _BLK, _OUT), lambda p, i: (0, (1 - p) * i, 0)),
            pl.BlockSpec((_BLK, _F), lambda p, i: ((1 - p) * i, 0)),
            pl.BlockSpec((1, _OUT), lambda p, i: (0, 0)),
            pl.BlockSpec((1, _OUT), lambda p, i: (0, 0)),
        ],
        out_specs=pl.BlockSpec((_BLK, _OUT), lambda p, i: (i, 0)),
        out_shape=jax.ShapeDtypeStruct((_B, _OUT), jnp.float32),
        scratch_shapes=[
            pltpu.VMEM((_B, _OUT), jnp.float32),
            pltpu.VMEM((8, _OUT), jnp.float32),
        ],
    )(g, par_t, gamma.reshape(1, _OUT), beta.reshape(1, _OUT))


def kernel(x, tables, W, b, gamma, beta):
    del b
    tabT = jnp.transpose(tables, (0, 2, 1))
    W3 = W.reshape(_F, _E, _OUT)
    P = _tc_project(tabT, W3)
    par = lax.shift_right_logical(x, 11) & 1
    idx2 = lax.shift_right_logical(x, 12) * _VB2 + (x & (_VB2 - 1))
    x3 = idx2.reshape(_F, _B // _BLK, _BLK // 128, 128)
    g = _sc_gather(P, x3)
    return _tc_sum_bn(g, par.T, gamma, beta)


# speedup vs baseline: 1.2744x; 1.2744x over previous
import functools

import jax
import jax.numpy as jnp
from jax import lax
from jax.experimental import pallas as pl
from jax.experimental.pallas import tpu as pltpu
from jax.experimental.pallas import tpu_sc as plsc

_F = 26
_V = 100001
_E = 50
_B = 16384
_OUT = 128
_EPS = 1e-5
_BLK = 512
_VB = 8192
_VP = -(-_V // _VB) * _VB


def _tc_project(tabT, W3):
    def body(t_ref, w_ref, p_ref):
        p = lax.dot_general(
            t_ref[0],
            w_ref[0],
            (((0,), (0,)), ((), ())),
            preferred_element_type=jnp.float32,
        )
        p_ref[0] = p

    return pl.pallas_call(
        body,
        grid=(_F, _VP // _VB),
        in_specs=[
            pl.BlockSpec((1, _E, _VB), lambda f, v: (f, 0, v)),
            pl.BlockSpec((1, _E, _OUT), lambda f, v: (f, 0, 0)),
        ],
        out_specs=pl.BlockSpec((1, _VB, _OUT), lambda f, v: (f, v, 0)),
        out_shape=jax.ShapeDtypeStruct((_F, _VP, _OUT), jnp.float32),
    )(tabT, W3)


def _sc_gather(p_hbm_arr, x3):
    info = plsc.get_sparse_core_info()
    nc, ns = info.num_cores, info.num_subcores
    nw = nc * ns
    bpw = _B // nw
    nch = bpw // 128

    @functools.partial(
        pl.kernel,
        mesh=plsc.VectorSubcoreMesh(core_axis_name="c", subcore_axis_name="s"),
        out_type=jax.ShapeDtypeStruct((_F, _B, _OUT), jnp.float32),
        scratch_types=[
            pltpu.VMEM((nch, 128), jnp.int32),
            pltpu.VMEM((bpw, _OUT), jnp.float32),
            pltpu.SemaphoreType.DMA,
        ],
    )
    def gather_kernel(p_hbm, x_hbm, out_hbm, idx_v, rows_v, sem):
        wid = lax.axis_index("s") * nc + lax.axis_index("c")
        base = wid * bpw

        def field_body(f, carry):
            pltpu.sync_copy(x_hbm.at[f, wid], idx_v)

            def gdesc(c):
                return pltpu.make_async_copy(
                    p_hbm.at[f].at[idx_v.at[c]],
                    rows_v.at[pl.ds(c * 128, 128)],
                    sem,
                )

            for c in range(nch):
                gdesc(c).start()
            for c in range(nch):
                gdesc(c).wait()
            pltpu.sync_copy(rows_v, out_hbm.at[f, pl.ds(base, bpw), :])
            return carry

        lax.fori_loop(0, _F, field_body, 0)

    return gather_kernel(p_hbm_arr, x3)


def _tc_sum_bn(g, gamma, beta):
    nb = _B // _BLK

    def body(g_ref, gm_ref, bt_ref, out_ref, y_buf, acc):
        p = pl.program_id(0)
        i = pl.program_id(1)

        @pl.when(p == 0)
        def _phase0():
            y = jnp.sum(g_ref[...], axis=0)
            y_buf[pl.ds(i * _BLK, _BLK), :] = y

            @pl.when(i == 0)
            def _init():
                acc[...] = jnp.zeros_like(acc)

            acc[0:1, :] = acc[0:1, :] + jnp.sum(y, axis=0, keepdims=True)
            acc[1:2, :] = acc[1:2, :] + jnp.sum(y * y, axis=0, keepdims=True)

        @pl.when(p == 1)
        def _phase1():
            mean = acc[0:1, :] * (1.0 / _B)
            var = acc[1:2, :] * (1.0 / _B) - mean * mean
            scale = gm_ref[...] * lax.rsqrt(var + _EPS)
            shift = bt_ref[...] - mean * scale
            out_ref[...] = y_buf[pl.ds(i * _BLK, _BLK), :] * scale + shift

    return pl.pallas_call(
        body,
        grid=(2, nb),
        in_specs=[
            pl.BlockSpec((_F, _BLK, _OUT), lambda p, i: (0, (1 - p) * i, 0)),
            pl.BlockSpec((1, _OUT), lambda p, i: (0, 0)),
            pl.BlockSpec((1, _OUT), lambda p, i: (0, 0)),
        ],
        out_specs=pl.BlockSpec((_BLK, _OUT), lambda p, i: (i, 0)),
        out_shape=jax.ShapeDtypeStruct((_B, _OUT), jnp.float32),
        scratch_shapes=[
            pltpu.VMEM((_B, _OUT), jnp.float32),
            pltpu.VMEM((8, _OUT), jnp.float32),
        ],
    )(g, gamma.reshape(1, _OUT), beta.reshape(1, _OUT))


def kernel(x, tables, W, b, gamma, beta):
    del b
    tabT = jnp.transpose(tables, (0, 2, 1))
    W3 = W.reshape(_F, _E, _OUT)
    P = _tc_project(tabT, W3)
    x3 = x.reshape(_F, _B // _BLK, (_B // (_B // _BLK)) // 128, 128)
    g = _sc_gather(P, x3)
    return _tc_sum_bn(g, gamma, beta)


# speedup vs baseline: 1.3426x; 1.0535x over previous
import functools

import jax
import jax.numpy as jnp
from jax import lax
from jax.experimental import pallas as pl
from jax.experimental.pallas import tpu as pltpu
from jax.experimental.pallas import tpu_sc as plsc

_F = 26
_V = 100001
_E = 50
_B = 16384
_OUT = 128
_EPS = 1e-5
_BLK = 512
_VB = 16384
_VP = -(-_V // _VB) * _VB


def _tc_project(tabT, W3):
    def body(t_ref, w_ref, p_ref):
        p = lax.dot_general(
            t_ref[0],
            w_ref[0],
            (((0,), (0,)), ((), ())),
            preferred_element_type=jnp.float32,
        )
        p_ref[0] = p

    return pl.pallas_call(
        body,
        grid=(_F, _VP // _VB),
        in_specs=[
            pl.BlockSpec((1, _E, _VB), lambda f, v: (f, 0, v)),
            pl.BlockSpec((1, _E, _OUT), lambda f, v: (f, 0, 0)),
        ],
        out_specs=pl.BlockSpec((1, _VB, _OUT), lambda f, v: (f, v, 0)),
        out_shape=jax.ShapeDtypeStruct((_F, _VP, _OUT), jnp.float32),
    )(tabT, W3)


def _sc_gather(p_hbm_arr, x3):
    info = plsc.get_sparse_core_info()
    nc, ns = info.num_cores, info.num_subcores
    nw = nc * ns
    bpw = _B // nw
    nch = bpw // 128

    @functools.partial(
        pl.kernel,
        mesh=plsc.VectorSubcoreMesh(core_axis_name="c", subcore_axis_name="s"),
        out_type=jax.ShapeDtypeStruct((_F, _B, _OUT), jnp.float32),
        scratch_types=[
            pltpu.VMEM((nch, 128), jnp.int32),
            pltpu.VMEM((bpw, _OUT), jnp.float32),
            pltpu.SemaphoreType.DMA,
        ],
    )
    def gather_kernel(p_hbm, x_hbm, out_hbm, idx_v, rows_v, sem):
        wid = lax.axis_index("s") * nc + lax.axis_index("c")
        base = wid * bpw

        def field_body(f, carry):
            pltpu.sync_copy(x_hbm.at[f, wid], idx_v)

            def gdesc(c):
                return pltpu.make_async_copy(
                    p_hbm.at[f].at[idx_v.at[c]],
                    rows_v.at[pl.ds(c * 128, 128)],
                    sem,
                )

            for c in range(nch):
                gdesc(c).start()
            for c in range(nch):
                gdesc(c).wait()
            pltpu.sync_copy(rows_v, out_hbm.at[f, pl.ds(base, bpw), :])
            return carry

        lax.fori_loop(0, _F, field_body, 0)

    return gather_kernel(p_hbm_arr, x3)


def _tc_sum_bn(g, gamma, beta):
    nb = _B // _BLK

    def body(g_ref, gm_ref, bt_ref, out_ref, y_buf, acc):
        p = pl.program_id(0)
        i = pl.program_id(1)

        @pl.when(p == 0)
        def _phase0():
            y = jnp.sum(g_ref[...], axis=0)
            y_buf[pl.ds(i * _BLK, _BLK), :] = y

            @pl.when(i == 0)
            def _init():
                acc[...] = jnp.zeros_like(acc)

            acc[0:1, :] = acc[0:1, :] + jnp.sum(y, axis=0, keepdims=True)
            acc[1:2, :] = acc[1:2, :] + jnp.sum(y * y, axis=0, keepdims=True)

        @pl.when(p == 1)
        def _phase1():
            mean = acc[0:1, :] * (1.0 / _B)
            var = acc[1:2, :] * (1.0 / _B) - mean * mean
            scale = gm_ref[...] * lax.rsqrt(var + _EPS)
            shift = bt_ref[...] - mean * scale
            out_ref[...] = y_buf[pl.ds(i * _BLK, _BLK), :] * scale + shift

    return pl.pallas_call(
        body,
        grid=(2, nb),
        in_specs=[
            pl.BlockSpec((_F, _BLK, _OUT), lambda p, i: (0, (1 - p) * i, 0)),
            pl.BlockSpec((1, _OUT), lambda p, i: (0, 0)),
            pl.BlockSpec((1, _OUT), lambda p, i: (0, 0)),
        ],
        out_specs=pl.BlockSpec((_BLK, _OUT), lambda p, i: (i, 0)),
        out_shape=jax.ShapeDtypeStruct((_B, _OUT), jnp.float32),
        scratch_shapes=[
            pltpu.VMEM((_B, _OUT), jnp.float32),
            pltpu.VMEM((8, _OUT), jnp.float32),
        ],
    )(g, gamma.reshape(1, _OUT), beta.reshape(1, _OUT))


def kernel(x, tables, W, b, gamma, beta):
    del b
    tabT = jnp.transpose(tables, (0, 2, 1))
    W3 = W.reshape(_F, _E, _OUT)
    P = _tc_project(tabT, W3)
    x3 = x.reshape(_F, _B // _BLK, (_B // (_B // _BLK)) // 128, 128)
    g = _sc_gather(P, x3)
    return _tc_sum_bn(g, gamma, beta)


# speedup vs baseline: 1.4444x; 1.0759x over previous
import functools

import jax
import jax.numpy as jnp
from jax import lax
from jax.experimental import pallas as pl
from jax.experimental.pallas import tpu as pltpu
from jax.experimental.pallas import tpu_sc as plsc

_F = 26
_V = 100001
_E = 50
_B = 16384
_OUT = 128
_EPS = 1e-5
_BLK = 512
_VB = 20480
_VP = -(-_V // _VB) * _VB


def _tc_project(tabT, W3):
    def body(t_ref, w_ref, p_ref):
        p = lax.dot_general(
            t_ref[0],
            w_ref[0],
            (((0,), (0,)), ((), ())),
            preferred_element_type=jnp.float32,
        )
        p_ref[0] = p

    return pl.pallas_call(
        body,
        grid=(_F, _VP // _VB),
        in_specs=[
            pl.BlockSpec((1, _E, _VB), lambda f, v: (f, 0, v)),
            pl.BlockSpec((1, _E, _OUT), lambda f, v: (f, 0, 0)),
        ],
        out_specs=pl.BlockSpec((1, _VB, _OUT), lambda f, v: (f, v, 0)),
        out_shape=jax.ShapeDtypeStruct((_F, _VP, _OUT), jnp.float32),
    )(tabT, W3)


def _sc_gather(p_hbm_arr, x3):
    info = plsc.get_sparse_core_info()
    nc, ns = info.num_cores, info.num_subcores
    nw = nc * ns
    bpw = _B // nw
    nch = bpw // 128

    @functools.partial(
        pl.kernel,
        mesh=plsc.VectorSubcoreMesh(core_axis_name="c", subcore_axis_name="s"),
        out_type=jax.ShapeDtypeStruct((_F, _B, _OUT), jnp.float32),
        scratch_types=[
            pltpu.VMEM((nch, 128), jnp.int32),
            pltpu.VMEM((bpw, _OUT), jnp.float32),
            pltpu.SemaphoreType.DMA,
        ],
    )
    def gather_kernel(p_hbm, x_hbm, out_hbm, idx_v, rows_v, sem):
        wid = lax.axis_index("s") * nc + lax.axis_index("c")
        base = wid * bpw

        def field_body(f, carry):
            pltpu.sync_copy(x_hbm.at[f, wid], idx_v)

            def gdesc(c):
                return pltpu.make_async_copy(
                    p_hbm.at[f].at[idx_v.at[c]],
                    rows_v.at[pl.ds(c * 128, 128)],
                    sem,
                )

            for c in range(nch):
                gdesc(c).start()
            for c in range(nch):
                gdesc(c).wait()
            pltpu.sync_copy(rows_v, out_hbm.at[f, pl.ds(base, bpw), :])
            return carry

        lax.fori_loop(0, _F, field_body, 0)

    return gather_kernel(p_hbm_arr, x3)


def _tc_sum_bn(g, gamma, beta):
    nb = _B // _BLK

    def body(g_ref, gm_ref, bt_ref, out_ref, y_buf, acc):
        p = pl.program_id(0)
        i = pl.program_id(1)

        @pl.when(p == 0)
        def _phase0():
            y = jnp.sum(g_ref[...], axis=0)
            y_buf[pl.ds(i * _BLK, _BLK), :] = y

            @pl.when(i == 0)
            def _init():
                acc[...] = jnp.zeros_like(acc)

            acc[0:1, :] = acc[0:1, :] + jnp.sum(y, axis=0, keepdims=True)
            acc[1:2, :] = acc[1:2, :] + jnp.sum(y * y, axis=0, keepdims=True)

        @pl.when(p == 1)
        def _phase1():
            mean = acc[0:1, :] * (1.0 / _B)
            var = acc[1:2, :] * (1.0 / _B) - mean * mean
            scale = gm_ref[...] * lax.rsqrt(var + _EPS)
            shift = bt_ref[...] - mean * scale
            out_ref[...] = y_buf[pl.ds(i * _BLK, _BLK), :] * scale + shift

    return pl.pallas_call(
        body,
        grid=(2, nb),
        in_specs=[
            pl.BlockSpec((_F, _BLK, _OUT), lambda p, i: (0, (1 - p) * i, 0)),
            pl.BlockSpec((1, _OUT), lambda p, i: (0, 0)),
            pl.BlockSpec((1, _OUT), lambda p, i: (0, 0)),
        ],
        out_specs=pl.BlockSpec((_BLK, _OUT), lambda p, i: (i, 0)),
        out_shape=jax.ShapeDtypeStruct((_B, _OUT), jnp.float32),
        scratch_shapes=[
            pltpu.VMEM((_B, _OUT), jnp.float32),
            pltpu.VMEM((8, _OUT), jnp.float32),
        ],
    )(g, gamma.reshape(1, _OUT), beta.reshape(1, _OUT))


def kernel(x, tables, W, b, gamma, beta):
    del b
    tabT = jnp.transpose(tables, (0, 2, 1))
    W3 = W.reshape(_F, _E, _OUT)
    P = _tc_project(tabT, W3)
    x3 = x.reshape(_F, _B // _BLK, (_B // (_B // _BLK)) // 128, 128)
    g = _sc_gather(P, x3)
    return _tc_sum_bn(g, gamma, beta)


# speedup vs baseline: 1.4560x; 1.0080x over previous
import functools

import jax
import jax.numpy as jnp
from jax import lax
from jax.experimental import pallas as pl
from jax.experimental.pallas import tpu as pltpu
from jax.experimental.pallas import tpu_sc as plsc

_F = 26
_V = 100001
_E = 50
_B = 16384
_OUT = 128
_EPS = 1e-5
_BLK = 512
_VB = 25600
_VP = -(-_V // _VB) * _VB


def _tc_project(tabT, W3):
    def body(t_ref, w_ref, p_ref):
        p = lax.dot_general(
            t_ref[0],
            w_ref[0],
            (((0,), (0,)), ((), ())),
            preferred_element_type=jnp.float32,
        )
        p_ref[0] = p

    return pl.pallas_call(
        body,
        grid=(_F, _VP // _VB),
        in_specs=[
            pl.BlockSpec((1, _E, _VB), lambda f, v: (f, 0, v)),
            pl.BlockSpec((1, _E, _OUT), lambda f, v: (f, 0, 0)),
        ],
        out_specs=pl.BlockSpec((1, _VB, _OUT), lambda f, v: (f, v, 0)),
        out_shape=jax.ShapeDtypeStruct((_F, _VP, _OUT), jnp.float32),
    )(tabT, W3)


def _sc_gather(p_hbm_arr, x3):
    info = plsc.get_sparse_core_info()
    nc, ns = info.num_cores, info.num_subcores
    nw = nc * ns
    bpw = _B // nw
    nch = bpw // 128

    @functools.partial(
        pl.kernel,
        mesh=plsc.VectorSubcoreMesh(core_axis_name="c", subcore_axis_name="s"),
        out_type=jax.ShapeDtypeStruct((_F, _B, _OUT), jnp.float32),
        scratch_types=[
            pltpu.VMEM((nch, 128), jnp.int32),
            pltpu.VMEM((bpw, _OUT), jnp.float32),
            pltpu.SemaphoreType.DMA,
        ],
    )
    def gather_kernel(p_hbm, x_hbm, out_hbm, idx_v, rows_v, sem):
        wid = lax.axis_index("s") * nc + lax.axis_index("c")
        base = wid * bpw

        def field_body(f, carry):
            pltpu.sync_copy(x_hbm.at[f, wid], idx_v)

            def gdesc(c):
                return pltpu.make_async_copy(
                    p_hbm.at[f].at[idx_v.at[c]],
                    rows_v.at[pl.ds(c * 128, 128)],
                    sem,
                )

            for c in range(nch):
                gdesc(c).start()
            for c in range(nch):
                gdesc(c).wait()
            pltpu.sync_copy(rows_v, out_hbm.at[f, pl.ds(base, bpw), :])
            return carry

        lax.fori_loop(0, _F, field_body, 0)

    return gather_kernel(p_hbm_arr, x3)


def _tc_sum_bn(g, gamma, beta):
    nb = _B // _BLK

    def body(g_ref, gm_ref, bt_ref, out_ref, y_buf, acc):
        p = pl.program_id(0)
        i = pl.program_id(1)

        @pl.when(p == 0)
        def _phase0():
            y = jnp.sum(g_ref[...], axis=0)
            y_buf[pl.ds(i * _BLK, _BLK), :] = y

            @pl.when(i == 0)
            def _init():
                acc[...] = jnp.zeros_like(acc)

            acc[0:1, :] = acc[0:1, :] + jnp.sum(y, axis=0, keepdims=True)
            acc[1:2, :] = acc[1:2, :] + jnp.sum(y * y, axis=0, keepdims=True)

        @pl.when(p == 1)
        def _phase1():
            mean = acc[0:1, :] * (1.0 / _B)
            var = acc[1:2, :] * (1.0 / _B) - mean * mean
            scale = gm_ref[...] * lax.rsqrt(var + _EPS)
            shift = bt_ref[...] - mean * scale
            out_ref[...] = y_buf[pl.ds(i * _BLK, _BLK), :] * scale + shift

    return pl.pallas_call(
        body,
        grid=(2, nb),
        in_specs=[
            pl.BlockSpec((_F, _BLK, _OUT), lambda p, i: (0, (1 - p) * i, 0)),
            pl.BlockSpec((1, _OUT), lambda p, i: (0, 0)),
            pl.BlockSpec((1, _OUT), lambda p, i: (0, 0)),
        ],
        out_specs=pl.BlockSpec((_BLK, _OUT), lambda p, i: (i, 0)),
        out_shape=jax.ShapeDtypeStruct((_B, _OUT), jnp.float32),
        scratch_shapes=[
            pltpu.VMEM((_B, _OUT), jnp.float32),
            pltpu.VMEM((8, _OUT), jnp.float32),
        ],
    )(g, gamma.reshape(1, _OUT), beta.reshape(1, _OUT))


def kernel(x, tables, W, b, gamma, beta):
    del b
    tabT = jnp.transpose(tables, (0, 2, 1))
    W3 = W.reshape(_F, _E, _OUT)
    P = _tc_project(tabT, W3)
    x3 = x.reshape(_F, _B // _BLK, (_B // (_B // _BLK)) // 128, 128)
    g = _sc_gather(P, x3)
    return _tc_sum_bn(g, gamma, beta)
